# baseline (speedup 1.0000x reference)
import functools

import jax
import jax.numpy as jnp
from jax import lax
from jax.experimental import pallas as pl
from jax.experimental.pallas import tpu as pltpu
from jax.experimental.pallas import tpu_sc as plsc

_ALPHA = 4
_NC = 2
_NS = 16


def _tc_copy_body(x_ref, o_ref):
    o_ref[...] = x_ref[...]


def _sc_gather_body(T, S, frames_hbm, out_hbm, buf0, buf1, sem0, sem1):
    c = lax.axis_index("c")
    s = lax.axis_index("s")
    w = c * _NS + s
    b = w // S
    j = w % S
    t = (j * (T - 1)) // (S - 1)

    nch = frames_hbm.shape[2]
    bufs = (buf0, buf1)
    sems = (sem0, sem1)
    copies = [None, None]
    copies[0] = pltpu.make_async_copy(frames_hbm.at[b, t, 0], bufs[0], sems[0])
    copies[0].start()
    for k in range(nch):
        nk = k + 1
        if nk < nch:
            copies[nk % 2] = pltpu.make_async_copy(
                frames_hbm.at[b, t, nk], bufs[nk % 2], sems[nk % 2])
            copies[nk % 2].start()
        copies[k % 2].wait()
        pltpu.sync_copy(bufs[k % 2], out_hbm.at[b, j, k])


def kernel(frames):
    B, T, C, H, W = frames.shape
    S = T // _ALPHA
    assert B * S == _NC * _NS, "one slow frame per SC vector subcore"
    import numpy as _np
    _expect = _np.linspace(0.0, T - 1, S).astype(_np.int32)
    _got = (_np.arange(S) * (T - 1)) // (S - 1)
    assert _np.array_equal(_expect, _got), (_expect, _got)

    slow = pl.kernel(
        functools.partial(_sc_gather_body, T, S),
        out_type=jax.ShapeDtypeStruct((B, S, C, H, W), jnp.float32),
        mesh=plsc.VectorSubcoreMesh(core_axis_name="c", subcore_axis_name="s"),
        scratch_types=[
            pltpu.VMEM((H, W), jnp.float32),
            pltpu.VMEM((H, W), jnp.float32),
            pltpu.SemaphoreType.DMA,
            pltpu.SemaphoreType.DMA,
        ],
    )(frames)

    blk_t = 4
    fast = pl.pallas_call(
        _tc_copy_body,
        grid=(B, T // blk_t),
        in_specs=[pl.BlockSpec((1, blk_t, C, H, W),
                               lambda b, i: (b, i, 0, 0, 0))],
        out_specs=pl.BlockSpec((1, blk_t, C, H, W),
                               lambda b, i: (b, i, 0, 0, 0)),
        out_shape=jax.ShapeDtypeStruct((B, T, C, H, W), jnp.float32),
    )(frames)

    return (slow, fast)


# speedup vs baseline: 1.0437x; 1.0437x over previous
import functools

import jax
import jax.numpy as jnp
from jax import lax
from jax.experimental import pallas as pl
from jax.experimental.pallas import tpu as pltpu
from jax.experimental.pallas import tpu_sc as plsc

_ALPHA = 4
_NC = 2
_NS = 16


def _tc_copy_body(x_ref, o_ref):
    o_ref[...] = x_ref[...]


def _sc_gather_body(T, S, frames_hbm, out_hbm, buf0, buf1, sem0, sem1):
    c = lax.axis_index("c")
    s = lax.axis_index("s")
    w = c * _NS + s
    b = w // S
    j = w % S
    t = (j * (T - 1)) // (S - 1)

    nch = frames_hbm.shape[2]
    bufs = (buf0, buf1)
    sems = (sem0, sem1)
    copies = [None, None]
    copies[0] = pltpu.make_async_copy(frames_hbm.at[b, t, 0], bufs[0], sems[0])
    copies[0].start()
    for k in range(nch):
        nk = k + 1
        if nk < nch:
            copies[nk % 2] = pltpu.make_async_copy(
                frames_hbm.at[b, t, nk], bufs[nk % 2], sems[nk % 2])
            copies[nk % 2].start()
        copies[k % 2].wait()
        pltpu.sync_copy(bufs[k % 2], out_hbm.at[b, j, k])


def kernel(frames):
    B, T, C, H, W = frames.shape
    S = T // _ALPHA
    assert B * S == _NC * _NS, "one slow frame per SC vector subcore"
    import numpy as _np
    _expect = _np.linspace(0.0, T - 1, S).astype(_np.int32)
    _got = (_np.arange(S) * (T - 1)) // (S - 1)
    assert _np.array_equal(_expect, _got), (_expect, _got)

    slow = pl.kernel(
        functools.partial(_sc_gather_body, T, S),
        out_type=jax.ShapeDtypeStruct((B, S, C, H, W), jnp.float32),
        mesh=plsc.VectorSubcoreMesh(core_axis_name="c", subcore_axis_name="s"),
        scratch_types=[
            pltpu.VMEM((H, W), jnp.float32),
            pltpu.VMEM((H, W), jnp.float32),
            pltpu.SemaphoreType.DMA,
            pltpu.SemaphoreType.DMA,
        ],
    )(frames)

    blk_t = 16
    fast = pl.pallas_call(
        _tc_copy_body,
        grid=(B, T // blk_t),
        in_specs=[pl.BlockSpec((1, blk_t, C, H, W),
                               lambda b, i: (b, i, 0, 0, 0))],
        out_specs=pl.BlockSpec((1, blk_t, C, H, W),
                               lambda b, i: (b, i, 0, 0, 0)),
        out_shape=jax.ShapeDtypeStruct((B, T, C, H, W), jnp.float32),
    )(frames)

    return (slow, fast)


# speedup vs baseline: 1.0483x; 1.0044x over previous
import functools

import jax
import jax.numpy as jnp
from jax import lax
from jax.experimental import pallas as pl
from jax.experimental.pallas import tpu as pltpu
from jax.experimental.pallas import tpu_sc as plsc

_ALPHA = 4
_NC = 2
_NS = 16


_NBUF = 6
_RAHEAD = 3


def _tc_copy_body(blk_t, x_hbm, o_hbm, *rest):
    B, T = x_hbm.shape[0], x_hbm.shape[1]
    units_per_b = T // blk_t
    n_units = B * units_per_b
    bufs = rest[:_NBUF]
    rsems = rest[_NBUF:2 * _NBUF]
    wsems = rest[2 * _NBUF:3 * _NBUF]

    def rd(u):
        b, t0 = u // units_per_b, (u % units_per_b) * blk_t
        return pltpu.make_async_copy(
            x_hbm.at[b, pl.ds(t0, blk_t)], bufs[u % _NBUF], rsems[u % _NBUF])

    def wr(u):
        b, t0 = u // units_per_b, (u % units_per_b) * blk_t
        return pltpu.make_async_copy(
            bufs[u % _NBUF], o_hbm.at[b, pl.ds(t0, blk_t)], wsems[u % _NBUF])

    for i in range(min(_RAHEAD, n_units)):
        rd(i).start()
    for u in range(n_units):
        rd(u).wait()
        wr(u).start()
        n = u + _RAHEAD
        if n < n_units:
            if n >= _NBUF:
                wr(n - _NBUF).wait()
            rd(n).start()
    for u in range(max(0, n_units - _NBUF), n_units):
        wr(u).wait()


def _sc_gather_body(T, S, frames_hbm, out_hbm, buf0, buf1, sem0, sem1):
    c = lax.axis_index("c")
    s = lax.axis_index("s")
    w = c * _NS + s
    b = w // S
    j = w % S
    t = (j * (T - 1)) // (S - 1)

    nch = frames_hbm.shape[2]
    bufs = (buf0, buf1)
    sems = (sem0, sem1)
    copies = [None, None]
    copies[0] = pltpu.make_async_copy(frames_hbm.at[b, t, 0], bufs[0], sems[0])
    copies[0].start()
    for k in range(nch):
        nk = k + 1
        if nk < nch:
            copies[nk % 2] = pltpu.make_async_copy(
                frames_hbm.at[b, t, nk], bufs[nk % 2], sems[nk % 2])
            copies[nk % 2].start()
        copies[k % 2].wait()
        pltpu.sync_copy(bufs[k % 2], out_hbm.at[b, j, k])


def kernel(frames):
    B, T, C, H, W = frames.shape
    S = T // _ALPHA
    assert B * S == _NC * _NS, "one slow frame per SC vector subcore"
    import numpy as _np
    _expect = _np.linspace(0.0, T - 1, S).astype(_np.int32)
    _got = (_np.arange(S) * (T - 1)) // (S - 1)
    assert _np.array_equal(_expect, _got), (_expect, _got)

    slow = pl.kernel(
        functools.partial(_sc_gather_body, T, S),
        out_type=jax.ShapeDtypeStruct((B, S, C, H, W), jnp.float32),
        mesh=plsc.VectorSubcoreMesh(core_axis_name="c", subcore_axis_name="s"),
        scratch_types=[
            pltpu.VMEM((H, W), jnp.float32),
            pltpu.VMEM((H, W), jnp.float32),
            pltpu.SemaphoreType.DMA,
            pltpu.SemaphoreType.DMA,
        ],
    )(frames)

    blk_t = 8
    fast = pl.pallas_call(
        functools.partial(_tc_copy_body, blk_t),
        in_specs=[pl.BlockSpec(memory_space=pltpu.HBM)],
        out_specs=pl.BlockSpec(memory_space=pltpu.HBM),
        out_shape=jax.ShapeDtypeStruct((B, T, C, H, W), jnp.float32),
        scratch_shapes=(
            [pltpu.VMEM((blk_t, C, H, W), jnp.float32)] * _NBUF
            + [pltpu.SemaphoreType.DMA] * (2 * _NBUF)
        ),
    )(frames)

    return (slow, fast)
